# Initial kernel scaffold; baseline (speedup 1.0000x reference)
import jax
import jax.numpy as jnp
from jax.experimental import pallas as pl


def kernel(x, targets, table):
    raise NotImplementedError("write your pallas kernel here")


# speedup vs baseline: 1.3141x; 1.3141x over previous
import functools

import jax
import jax.numpy as jnp
from jax import lax
from jax.experimental import pallas as pl
from jax.experimental.pallas import tpu as pltpu
from jax.experimental.pallas import tpu_sc as plsc

VOCAB = 1000
VPAD = 1024
B = 1024
T = 200
N = B * T

NC = 2
NS = 16
NW = NC * NS
ROWS_PER_W = N // NW
K = 32
NCHUNK = ROWS_PER_W // K
L = 16


def _lse_body(table_ref, out_ref):
    t = table_ref[...]
    m = jnp.max(t, axis=1)
    s = jnp.sum(jnp.exp(t - m[:, None]), axis=1)
    out_ref[...] = m + jnp.log(s)


def _loss_body(part_ref, out_ref):
    s = jnp.sum(part_ref[...])
    out_ref[...] = jnp.full((1, 1), 1.0 / N, jnp.float32) * s


def _sc_body(table, x3, t3, lse, out, part, idx_v, tgt_v, lse_v, buf, acc, sem):
    wid = lax.axis_index("s") * NC + lax.axis_index("c")
    pltpu.sync_copy(x3.at[wid], idx_v)
    pltpu.sync_copy(t3.at[wid], tgt_v)
    pltpu.sync_copy(lse, lse_v)
    acc[...] = jnp.zeros((L,), jnp.float32)
    base = wid * ROWS_PER_W

    def chunk(j, carry):
        pltpu.async_copy(table.at[idx_v.at[j]], buf, sem).wait()
        for b in range(K // L):
            iv = lax.iota(jnp.int32, L) + (b * L)
            xv = idx_v[j, pl.ds(b * L, L)]
            tv = tgt_v[j, pl.ds(b * L, L)]
            lv = plsc.load_gather(lse_v, [xv])
            pv = plsc.load_gather(buf, [iv, tv])
            acc[...] = acc[...] + (lv - pv)
        pltpu.sync_copy(buf, out.at[pl.ds(base + j * K, K)])
        return carry

    lax.fori_loop(0, NCHUNK, chunk, 0)
    pltpu.sync_copy(acc, part.at[wid])


def kernel(x, targets, table):
    x3 = x.reshape(NW, NCHUNK, K)
    t3 = targets.reshape(NW, NCHUNK, K)

    tpad = jnp.pad(table, ((0, VPAD - VOCAB), (0, VPAD - VOCAB)),
                   constant_values=-1e30)
    lse = pl.pallas_call(
        _lse_body,
        out_shape=jax.ShapeDtypeStruct((VPAD,), jnp.float32),
    )(tpad)

    mesh = plsc.VectorSubcoreMesh(core_axis_name="c", subcore_axis_name="s")
    run = functools.partial(
        pl.kernel,
        out_type=[
            jax.ShapeDtypeStruct((N, VOCAB), jnp.float32),
            jax.ShapeDtypeStruct((NW, L), jnp.float32),
        ],
        mesh=mesh,
        compiler_params=pltpu.CompilerParams(
            needs_layout_passes=False, use_tc_tiling_on_sc=False),
        scratch_types=[
            pltpu.VMEM((NCHUNK, K), jnp.int32),
            pltpu.VMEM((NCHUNK, K), jnp.int32),
            pltpu.VMEM((VPAD,), jnp.float32),
            pltpu.VMEM((K, VOCAB), jnp.float32),
            pltpu.VMEM((L,), jnp.float32),
            pltpu.SemaphoreType.DMA,
        ],
    )(_sc_body)
    logits, part = run(table, x3, t3, lse)

    loss2 = pl.pallas_call(
        _loss_body,
        out_shape=jax.ShapeDtypeStruct((1, 1), jnp.float32),
    )(part)
    return logits, loss2[0, 0]


# speedup vs baseline: 1.3813x; 1.0511x over previous
import functools

import jax
import jax.numpy as jnp
from jax import lax
from jax.experimental import pallas as pl
from jax.experimental.pallas import tpu as pltpu
from jax.experimental.pallas import tpu_sc as plsc

VOCAB = 1000
VPAD = 1024
B = 1024
T = 200
N = B * T

NC = 2
NS = 16
NW = NC * NS
ROWS_PER_W = N // NW
K = 32
NCHUNK = ROWS_PER_W // K
L = 16


def _lse_body(table_ref, out_ref):
    t = table_ref[...]
    m = jnp.max(t, axis=1)
    s = jnp.sum(jnp.exp(t - m[:, None]), axis=1)
    out_ref[...] = m + jnp.log(s)


def _loss_body(part_ref, out_ref):
    s = jnp.sum(part_ref[...])
    out_ref[...] = jnp.full((1, 1), 1.0 / N, jnp.float32) * s


def _sc_body(table, x3, t3, lse, out, part,
             idx_v, tgt_v, lse_v, buf0, buf1, acc, gs0, gs1, os0, os1):
    wid = lax.axis_index("s") * NC + lax.axis_index("c")
    pltpu.sync_copy(x3.at[wid], idx_v)
    pltpu.sync_copy(t3.at[wid], tgt_v)
    pltpu.sync_copy(lse, lse_v)
    acc[...] = jnp.zeros((L,), jnp.float32)
    base = wid * ROWS_PER_W

    def loss_terms(j, buf):
        for b in range(K // L):
            iv = lax.iota(jnp.int32, L) + (b * L)
            xv = idx_v[j, pl.ds(b * L, L)]
            tv = tgt_v[j, pl.ds(b * L, L)]
            lv = plsc.load_gather(lse_v, [xv])
            pv = plsc.load_gather(buf, [iv, tv])
            acc[...] = acc[...] + (lv - pv)

    pltpu.async_copy(table.at[idx_v.at[0]], buf0, gs0)
    pltpu.async_copy(table.at[idx_v.at[1]], buf1, gs1)

    def pair(i, carry):
        jA = 2 * i
        jB = jA + 1
        pltpu.make_async_copy(table.at[idx_v.at[jA]], buf0, gs0).wait()
        loss_terms(jA, buf0)
        hA = pltpu.async_copy(buf0, out.at[pl.ds(base + jA * K, K)], os0)
        pltpu.make_async_copy(table.at[idx_v.at[jB]], buf1, gs1).wait()
        loss_terms(jB, buf1)
        hB = pltpu.async_copy(buf1, out.at[pl.ds(base + jB * K, K)], os1)
        jC = jnp.minimum(jA + 2, NCHUNK - 1)
        jD = jnp.minimum(jB + 2, NCHUNK - 1)
        hA.wait()
        pltpu.async_copy(table.at[idx_v.at[jC]], buf0, gs0)
        hB.wait()
        pltpu.async_copy(table.at[idx_v.at[jD]], buf1, gs1)
        return carry

    lax.fori_loop(0, NCHUNK // 2, pair, 0)
    pltpu.make_async_copy(table.at[idx_v.at[NCHUNK - 1]], buf0, gs0).wait()
    pltpu.make_async_copy(table.at[idx_v.at[NCHUNK - 1]], buf1, gs1).wait()
    pltpu.sync_copy(acc, part.at[wid])


def kernel(x, targets, table):
    x3 = x.reshape(NW, NCHUNK, K)
    t3 = targets.reshape(NW, NCHUNK, K)

    tpad = jnp.pad(table, ((0, VPAD - VOCAB), (0, VPAD - VOCAB)),
                   constant_values=-1e30)
    lse = pl.pallas_call(
        _lse_body,
        out_shape=jax.ShapeDtypeStruct((VPAD,), jnp.float32),
    )(tpad)

    mesh = plsc.VectorSubcoreMesh(core_axis_name="c", subcore_axis_name="s")
    run = functools.partial(
        pl.kernel,
        out_type=[
            jax.ShapeDtypeStruct((N, VOCAB), jnp.float32),
            jax.ShapeDtypeStruct((NW, L), jnp.float32),
        ],
        mesh=mesh,
        compiler_params=pltpu.CompilerParams(
            needs_layout_passes=False, use_tc_tiling_on_sc=False),
        scratch_types=[
            pltpu.VMEM((NCHUNK, K), jnp.int32),
            pltpu.VMEM((NCHUNK, K), jnp.int32),
            pltpu.VMEM((VPAD,), jnp.float32),
            pltpu.VMEM((K, VOCAB), jnp.float32),
            pltpu.VMEM((K, VOCAB), jnp.float32),
            pltpu.VMEM((L,), jnp.float32),
            pltpu.SemaphoreType.DMA,
            pltpu.SemaphoreType.DMA,
            pltpu.SemaphoreType.DMA,
            pltpu.SemaphoreType.DMA,
        ],
    )(_sc_body)
    logits, part = run(table, x3, t3, lse)

    loss2 = pl.pallas_call(
        _loss_body,
        out_shape=jax.ShapeDtypeStruct((1, 1), jnp.float32),
    )(part)
    return logits, loss2[0, 0]


# speedup vs baseline: 2.1200x; 1.5348x over previous
import functools

import jax
import jax.numpy as jnp
from jax import lax
from jax.experimental import pallas as pl
from jax.experimental.pallas import tpu as pltpu
from jax.experimental.pallas import tpu_sc as plsc

VOCAB = 1000
VPAD = 1024
B = 1024
T = 200
N = B * T

NC = 2
NS = 16
NW = NC * NS
ROWS_PER_W = N // NW
K = 16
NCHUNK = ROWS_PER_W // K
L = 16
NSEG = VOCAB // L
TAIL = VOCAB - NSEG * L


def _lse_body(table_ref, out_ref):
    t = table_ref[...]
    m = jnp.max(t, axis=1)
    s = jnp.sum(jnp.exp(t - m[:, None]), axis=1)
    out_ref[...] = m + jnp.log(s)


def _loss_body(part_ref, out_ref):
    s = jnp.sum(part_ref[...])
    out_ref[...] = jnp.full((1, 1), 1.0 / N, jnp.float32) * s


def _sc_body(table, x2, t2, lse, out, part,
             idx_v, tgt_v, lse_v, buf0, buf1, pak0, pak1, acc,
             gs0, gs1, os0, os1):
    wid = lax.axis_index("s") * NC + lax.axis_index("c")
    pltpu.sync_copy(x2.at[wid], idx_v)
    pltpu.sync_copy(t2.at[wid], tgt_v)
    pltpu.sync_copy(lse, lse_v)
    acc[...] = jnp.zeros((L,), jnp.float32)
    base = wid * ROWS_PER_W

    def loss_terms(j, buf):
        iv = lax.iota(jnp.int32, L)
        xv = idx_v[pl.ds(j * K, L)]
        tv = tgt_v[pl.ds(j * K, L)]
        lv = plsc.load_gather(lse_v, [xv])
        pv = plsc.load_gather(buf, [iv, tv])
        acc[...] = acc[...] + (lv - pv)

    def repack(src, dst):
        def row(r, carry):
            for c in range(NSEG):
                dst[r, pl.ds(c * L, L)] = src[r, pl.ds(c * L, L)]
            seg = src[r, pl.ds(NSEG * L, L)]
            cols = NSEG * L + lax.iota(jnp.int32, L)
            rows = jnp.zeros((L,), jnp.int32) + r
            plsc.store_scatter(dst, [rows, cols], seg, mask=cols < VOCAB)
            return carry
        lax.fori_loop(0, K, row, 0)

    def gather_chunk(j, buf, sem):
        return pltpu.async_copy(table.at[idx_v.at[pl.ds(j * K, K)]], buf, sem)

    gather_chunk(0, buf0, gs0)
    gather_chunk(1, buf1, gs1)

    def pair(i, carry):
        jA = 2 * i
        jB = jA + 1
        pltpu.make_async_copy(table.at[idx_v.at[pl.ds(0, K)]], buf0, gs0).wait()
        loss_terms(jA, buf0)

        @pl.when(i > 0)
        def _():
            pltpu.make_async_copy(pak0, out.at[pl.ds(0, K)], os0).wait()

        repack(buf0, pak0)
        gather_chunk(jnp.minimum(jA + 2, NCHUNK - 1), buf0, gs0)
        pltpu.async_copy(pak0, out.at[pl.ds(base + jA * K, K)], os0)
        pltpu.make_async_copy(table.at[idx_v.at[pl.ds(0, K)]], buf1, gs1).wait()
        loss_terms(jB, buf1)

        @pl.when(i > 0)
        def _():
            pltpu.make_async_copy(pak1, out.at[pl.ds(0, K)], os1).wait()

        repack(buf1, pak1)
        gather_chunk(jnp.minimum(jB + 2, NCHUNK - 1), buf1, gs1)
        pltpu.async_copy(pak1, out.at[pl.ds(base + jB * K, K)], os1)
        return carry

    lax.fori_loop(0, NCHUNK // 2, pair, 0)
    pltpu.make_async_copy(pak0, out.at[pl.ds(0, K)], os0).wait()
    pltpu.make_async_copy(pak1, out.at[pl.ds(0, K)], os1).wait()
    pltpu.make_async_copy(table.at[idx_v.at[pl.ds(0, K)]], buf0, gs0).wait()
    pltpu.make_async_copy(table.at[idx_v.at[pl.ds(0, K)]], buf1, gs1).wait()
    pltpu.sync_copy(acc, part.at[wid])


def kernel(x, targets, table):
    x2 = x.reshape(NW, ROWS_PER_W)
    t2 = targets.reshape(NW, ROWS_PER_W)

    tpad = jnp.pad(table, ((0, VPAD - VOCAB), (0, VPAD - VOCAB)),
                   constant_values=-1e30)
    lse = pl.pallas_call(
        _lse_body,
        out_shape=jax.ShapeDtypeStruct((VPAD,), jnp.float32),
    )(tpad)

    mesh = plsc.VectorSubcoreMesh(core_axis_name="c", subcore_axis_name="s")
    run = functools.partial(
        pl.kernel,
        out_type=[
            jax.ShapeDtypeStruct((N, VOCAB), jnp.float32),
            jax.ShapeDtypeStruct((NW, L), jnp.float32),
        ],
        mesh=mesh,
        compiler_params=pltpu.CompilerParams(
            needs_layout_passes=False, use_tc_tiling_on_sc=True),
        scratch_types=[
            pltpu.VMEM((ROWS_PER_W,), jnp.int32),
            pltpu.VMEM((ROWS_PER_W,), jnp.int32),
            pltpu.VMEM((VPAD,), jnp.float32),
            pltpu.VMEM((K, VPAD), jnp.float32),
            pltpu.VMEM((K, VPAD), jnp.float32),
            pltpu.VMEM((K, VOCAB), jnp.float32),
            pltpu.VMEM((K, VOCAB), jnp.float32),
            pltpu.VMEM((L,), jnp.float32),
            pltpu.SemaphoreType.DMA,
            pltpu.SemaphoreType.DMA,
            pltpu.SemaphoreType.DMA,
            pltpu.SemaphoreType.DMA,
        ],
    )(_sc_body)
    logits, part = run(tpad, x2, t2, lse)

    loss2 = pl.pallas_call(
        _loss_body,
        out_shape=jax.ShapeDtypeStruct((1, 1), jnp.float32),
    )(part)
    return logits, loss2[0, 0]
